# baseline (speedup 1.0000x reference)
import functools

import jax
import jax.numpy as jnp
from jax import lax
from jax.experimental import pallas as pl
from jax.experimental.pallas import tpu as pltpu
from jax.experimental.pallas import tpu_sc as plsc

_E = 320000
_E_PAD = 327680
_NW = 32
_EPT = _E_PAD // _NW
_CH = 128
_NCH = _EPT // _CH
_D = 128

_N0, _K0, _K1 = 10000, 5000, 2500
_N0P, _N1P, _N2P = 10240, 5120, 2560


@functools.cache
def _make_scatter(n_pad):
    mesh = plsc.VectorSubcoreMesh(core_axis_name="c", subcore_axis_name="s")
    rpt = n_pad // 16

    @functools.partial(
        pl.kernel,
        out_type=jax.ShapeDtypeStruct((2, n_pad, _D), jnp.float32),
        mesh=mesh,
        scratch_types=[
            pltpu.VMEM((_CH,), jnp.int32),
            pltpu.VMEM((_CH,), jnp.int32),
            pltpu.VMEM((_CH, _D), jnp.float32),
            pltpu.VMEM((_CH,), jnp.int32),
            pltpu.VMEM((_CH,), jnp.int32),
            pltpu.VMEM((_CH, _D), jnp.float32),
            pltpu.VMEM((16,), jnp.int32),
            pltpu.VMEM_SHARED((n_pad, _D), jnp.float32),
            pltpu.SemaphoreType.DMA,
            pltpu.SemaphoreType.DMA,
        ],
        compiler_params=pltpu.CompilerParams(needs_layout_passes=False),
    )
    def k(hpre_hbm, src_hbm, dst_hbm, zeros_hbm, counts_hbm, out_hbm,
          giA, siA, rowsA, giB, siB, rowsB, cbuf, acc, semA, semB):
        cid = lax.axis_index("c")
        sid = lax.axis_index("s")
        wid = sid * 2 + cid
        pltpu.sync_copy(zeros_hbm, acc.at[pl.ds(sid * rpt, rpt)])
        pltpu.sync_copy(counts_hbm.at[wid], cbuf)
        npairs = cbuf[pl.ds(0, 16)][0]
        plsc.subcore_barrier()
        base = wid * _EPT

        pltpu.sync_copy(src_hbm.at[pl.ds(base, _CH)], giA)
        pltpu.sync_copy(dst_hbm.at[pl.ds(base, _CH)], siA)
        pltpu.async_copy(hpre_hbm.at[giA], rowsA, semA)

        def body(i, carry):
            offb = base + (2 * i + 1) * _CH
            pltpu.sync_copy(src_hbm.at[pl.ds(offb, _CH)], giB)
            pltpu.sync_copy(dst_hbm.at[pl.ds(offb, _CH)], siB)
            pltpu.async_copy(hpre_hbm.at[giB], rowsB, semB)
            pltpu.make_async_copy(hpre_hbm.at[giA], rowsA, semA).wait()
            pltpu.sync_copy(rowsA, acc.at[siA], add=True)

            @pl.when(i < npairs - 1)
            def _():
                offa = base + (2 * i + 2) * _CH
                pltpu.sync_copy(src_hbm.at[pl.ds(offa, _CH)], giA)
                pltpu.sync_copy(dst_hbm.at[pl.ds(offa, _CH)], siA)
                pltpu.async_copy(hpre_hbm.at[giA], rowsA, semA)

            pltpu.make_async_copy(hpre_hbm.at[giB], rowsB, semB).wait()
            pltpu.sync_copy(rowsB, acc.at[siB], add=True)
            return carry

        lax.fori_loop(0, npairs, body, 0)
        plsc.subcore_barrier()
        pltpu.sync_copy(
            acc.at[pl.ds(sid * rpt, rpt)],
            out_hbm.at[cid].at[pl.ds(sid * rpt, rpt)],
        )

    return k


@functools.cache
def _make_relabel(n_old_pad, n_new_pad, k_new):
    mesh = plsc.VectorSubcoreMesh(core_axis_name="c", subcore_axis_name="s")
    rh = n_new_pad // 128
    R = 2 * rh

    @functools.partial(
        pl.kernel,
        out_type=[
            jax.ShapeDtypeStruct((2, _E_PAD), jnp.int32),
            jax.ShapeDtypeStruct((_NW, 16), jnp.int32),
            jax.ShapeDtypeStruct((2, R, 128), jnp.float32),
        ],
        mesh=mesh,
        scratch_types=[
            pltpu.VMEM((n_old_pad,), jnp.int32),
            pltpu.VMEM((_EPT,), jnp.int32),
            pltpu.VMEM((_EPT,), jnp.int32),
            pltpu.VMEM((_EPT + 256,), jnp.int32),
            pltpu.VMEM((_EPT + 256,), jnp.int32),
            pltpu.VMEM((R, 128), jnp.float32),
            pltpu.VMEM((min(R, 128),), jnp.int32),
            pltpu.VMEM((max(R - 128, 8),), jnp.int32),
            pltpu.VMEM((16,), jnp.int32),
            pltpu.VMEM((16,), jnp.int32),
            pltpu.VMEM_SHARED((R, 128), jnp.float32),
        ],
        compiler_params=pltpu.CompilerParams(needs_layout_passes=False),
    )
    def k(map_hbm, s_hbm, d_hbm, zeros_hbm, iota_hbm, counts_in_hbm,
          edges_hbm, counts_hbm, deg_hbm,
          mapv, sin, din, sb, db, hist, ri0, ri1, cnt16, cin16, acc):
        cid = lax.axis_index("c")
        sid = lax.axis_index("s")
        wid = sid * 2 + cid
        base = wid * _EPT
        pltpu.sync_copy(map_hbm, mapv)
        pltpu.sync_copy(s_hbm.at[pl.ds(base, _EPT)], sin)
        pltpu.sync_copy(d_hbm.at[pl.ds(base, _EPT)], din)
        pltpu.sync_copy(counts_in_hbm.at[wid], cin16)
        pltpu.sync_copy(zeros_hbm, hist)

        @pl.when(sid == 0)
        def _():
            pltpu.sync_copy(zeros_hbm, acc)

        pltpu.sync_copy(iota_hbm.at[pl.ds(0, min(R, 128))], ri0)
        if R > 128:
            pltpu.sync_copy(iota_hbm.at[pl.ds(128, R - 128)], ri1)
        ones = jnp.ones((16,), jnp.float32)

        def body(i, cur):
            s16 = sin[pl.ds(i * 16, 16)]
            d16 = din[pl.ds(i * 16, 16)]
            ns = plsc.load_gather(mapv, [s16])
            nd = plsc.load_gather(mapv, [d16])
            m = (ns < k_new) & (nd < k_new)
            plsc.addupdate_scatter(
                hist, [lax.shift_right_logical(ns, 7), ns & 127], ones, mask=m)
            plsc.addupdate_scatter(
                hist, [lax.shift_right_logical(nd, 7) + rh, nd & 127], ones,
                mask=m)
            plsc.store_compressed(sb.at[pl.ds(cur, 16)], ns, mask=m)
            plsc.store_compressed(db.at[pl.ds(cur, 16)], nd, mask=m)
            return cur + plsc.all_reduce_population_count(m)[0]

        cur = lax.fori_loop(0, cin16[pl.ds(0, 16)][0] * 16, body, 0)
        lane = jnp.arange(16, dtype=jnp.int32)
        for j in range(16):
            sb[pl.ds(cur + j * 16, 16)] = jnp.zeros((16,), jnp.int32)
            db[pl.ds(cur + j * 16, 16)] = k_new + ((lane + j * 16) & 31)
        npairs = jnp.maximum(lax.shift_right_logical(cur + 255, 8), 1)
        cnt16[...] = jnp.full((16,), 0, jnp.int32) + npairs
        pltpu.sync_copy(cnt16, counts_hbm.at[wid])
        pltpu.sync_copy(sb.at[pl.ds(0, _EPT)],
                        edges_hbm.at[0].at[pl.ds(base, _EPT)])
        pltpu.sync_copy(db.at[pl.ds(0, _EPT)],
                        edges_hbm.at[1].at[pl.ds(base, _EPT)])
        plsc.subcore_barrier()
        if R <= 128:
            pltpu.sync_copy(hist, acc.at[ri0], add=True)
        else:
            pltpu.sync_copy(hist.at[pl.ds(0, 128)], acc.at[ri0], add=True)
            pltpu.sync_copy(hist.at[pl.ds(128, R - 128)], acc.at[ri1], add=True)
        plsc.subcore_barrier()

        @pl.when(sid == 0)
        def _():
            pltpu.sync_copy(acc, deg_hbm.at[cid])

    return k


@functools.cache
def _make_degrees(n_pad, n_real):
    mesh = plsc.VectorSubcoreMesh(core_axis_name="c", subcore_axis_name="s")
    rh = n_pad // 128
    R = 2 * rh

    @functools.partial(
        pl.kernel,
        out_type=jax.ShapeDtypeStruct((2, R, 128), jnp.float32),
        mesh=mesh,
        scratch_types=[
            pltpu.VMEM((_EPT,), jnp.int32),
            pltpu.VMEM((_EPT,), jnp.int32),
            pltpu.VMEM((R, 128), jnp.float32),
            pltpu.VMEM((min(R, 128),), jnp.int32),
            pltpu.VMEM((max(R - 128, 8),), jnp.int32),
            pltpu.VMEM_SHARED((R, 128), jnp.float32),
            pltpu.SemaphoreType.DMA,
        ],
        compiler_params=pltpu.CompilerParams(needs_layout_passes=False),
    )
    def k(src_hbm, dst_hbm, zeros_hbm, iota_hbm, out_hbm,
          si, di, hist, ri0, ri1, acc, sem):
        cid = lax.axis_index("c")
        sid = lax.axis_index("s")
        wid = sid * 2 + cid
        pltpu.sync_copy(zeros_hbm, hist)

        @pl.when(sid == 0)
        def _():
            pltpu.sync_copy(zeros_hbm, acc)

        base = wid * _EPT
        pltpu.sync_copy(src_hbm.at[pl.ds(base, _EPT)], si)
        pltpu.sync_copy(dst_hbm.at[pl.ds(base, _EPT)], di)
        ones = jnp.ones((16,), jnp.float32)

        def body(i, carry):
            s16 = si[pl.ds(i * 16, 16)]
            d16 = di[pl.ds(i * 16, 16)]
            m = (s16 < n_real) & (d16 < n_real)
            plsc.addupdate_scatter(
                hist, [lax.shift_right_logical(s16, 7), s16 & 127], ones, mask=m)
            plsc.addupdate_scatter(
                hist, [lax.shift_right_logical(d16, 7) + rh, d16 & 127], ones, mask=m)
            return carry

        lax.fori_loop(0, _EPT // 16, body, 0)
        pltpu.sync_copy(iota_hbm.at[pl.ds(0, min(R, 128))], ri0)
        if R > 128:
            pltpu.sync_copy(iota_hbm.at[pl.ds(128, R - 128)], ri1)
        plsc.subcore_barrier()
        if R <= 128:
            pltpu.sync_copy(hist, acc.at[ri0], add=True)
        else:
            pltpu.sync_copy(hist.at[pl.ds(0, 128)], acc.at[ri0], add=True)
            pltpu.sync_copy(hist.at[pl.ds(128, R - 128)], acc.at[ri1], add=True)
        plsc.subcore_barrier()

        @pl.when(sid == 0)
        def _():
            pltpu.sync_copy(acc, out_hbm.at[cid])

    return k


def _degrees_sc(s, d, n_pad, n_real):
    R = 2 * n_pad // 128
    zeros = jnp.zeros((R, 128), jnp.float32)
    iota = jnp.arange(R, dtype=jnp.int32)
    parts = _make_degrees(n_pad, n_real)(s, d, zeros, iota)
    deg2 = (parts[0] + parts[1]).reshape(2, n_pad)
    return deg2[0], deg2[1]


@functools.cache
def _make_dense(n_pad, with_scores):
    br = 256
    grid = (n_pad // br,)

    def body(agg_ref, ni_ref, w_ref, b_ref, p_ref, h_ref, s_ref):
        a = (agg_ref[0] + agg_ref[1]) * ni_ref[...]
        h = jnp.dot(a, w_ref[...], preferred_element_type=jnp.float32)
        h = jnp.maximum(h + b_ref[...], 0.0)
        h_ref[...] = h
        if with_scores:
            s_ref[...] = jnp.dot(h, p_ref[...], preferred_element_type=jnp.float32)

    def body_ns(agg_ref, ni_ref, w_ref, b_ref, p_ref, h_ref):
        body(agg_ref, ni_ref, w_ref, b_ref, p_ref, h_ref, None)

    out_shape = [jax.ShapeDtypeStruct((n_pad, _D), jnp.float32)]
    out_specs = [pl.BlockSpec((br, _D), lambda i: (i, 0))]
    if with_scores:
        out_shape.append(jax.ShapeDtypeStruct((n_pad, 1), jnp.float32))
        out_specs.append(pl.BlockSpec((br, 1), lambda i: (i, 0)))

    return pl.pallas_call(
        body if with_scores else body_ns,
        grid=grid,
        in_specs=[
            pl.BlockSpec((2, br, _D), lambda i: (0, i, 0)),
            pl.BlockSpec((br, 1), lambda i: (i, 0)),
            pl.BlockSpec((_D, _D), lambda i: (0, 0)),
            pl.BlockSpec((1, _D), lambda i: (0, 0)),
            pl.BlockSpec((_D, 1), lambda i: (0, 0)),
        ],
        out_specs=out_specs,
        out_shape=out_shape,
    )


def _deg_to_norms(parts, n_pad):
    deg2 = (parts[0] + parts[1]).reshape(2, n_pad)
    dego, degi = deg2[0], deg2[1]
    norm_out = jnp.where(dego > 0, lax.rsqrt(jnp.maximum(dego, 1.0)), 0.0)
    norm_in = jnp.where(degi > 0, lax.rsqrt(jnp.maximum(degi, 1.0)), 0.0)
    return norm_out, norm_in


def _norms(s, d, n_pad, n_real):
    R = 2 * n_pad // 128
    zeros = jnp.zeros((R, 128), jnp.float32)
    iota = jnp.arange(R, dtype=jnp.int32)
    parts = _make_degrees(n_pad, n_real)(s, d, zeros, iota)
    return _deg_to_norms(parts, n_pad)


def _relabel(map_ext, s, d, counts_in, n_old_pad, n_new_pad, k_new):
    R = 2 * n_new_pad // 128
    zeros = jnp.zeros((R, 128), jnp.float32)
    iota = jnp.arange(R, dtype=jnp.int32)
    edges, counts, degp = _make_relabel(n_old_pad, n_new_pad, k_new)(
        map_ext, s, d, zeros, iota, counts_in)
    no, ni = _deg_to_norms(degp, n_new_pad)
    return edges[0], edges[1], counts, no, ni


def _conv(h, s, d, counts, n_pad, W, b, P, norm_out, norm_in):
    h_pre = h * norm_out[:, None]
    zeros = jnp.zeros((n_pad // 16, _D), jnp.float32)
    agg2 = _make_scatter(n_pad)(h_pre, s, d, zeros, counts)
    dense = _make_dense(n_pad, P is not None)
    pdummy = jnp.zeros((_D, 1), jnp.float32) if P is None else P
    outs = dense(agg2, norm_in[:, None], W, b.reshape(1, _D), pdummy)
    if P is None:
        return outs[0], None
    return outs[0], outs[1][:, 0]


def _pad_rows(h, n_pad):
    return jnp.pad(h, ((0, n_pad - h.shape[0]), (0, 0)))


def _trash_map(ids, k_new, n_old_pad, n_new_pad):
    base = k_new + (jnp.arange(n_old_pad, dtype=jnp.int32) % (n_new_pad - k_new))
    return base.at[ids].set(jnp.arange(k_new, dtype=jnp.int32))


def kernel(x, edge_index, W0, b0, W1, b1, W2, b2, W3, b3, W4, b4, P0, pb0, P1, pb1):
    src = edge_index[0]
    dst = edge_index[1]
    pad = _N0 + (jnp.arange(_E_PAD - _E, dtype=jnp.int32) % (_N0P - _N0))
    s0 = jnp.concatenate([src, pad])
    d0 = jnp.concatenate([dst, pad])

    cnt_full = jnp.full((_NW, 16), _NCH // 2, jnp.int32)

    no0, ni0 = _norms(s0, d0, _N0P, _N0)
    h, lg0 = _conv(_pad_rows(x, _N0P), s0, d0, cnt_full, _N0P, W0, b0, P0,
                   no0, ni0)
    hid0 = h
    _, ids0 = lax.top_k(lg0[:_N0], _K0)
    sc0 = jax.nn.sigmoid(lg0[ids0] + pb0[0])
    map0 = _trash_map(ids0, _K0, _N0P, _N1P)
    s1, d1, cnt1, no1, ni1 = _relabel(map0, s0, d0, cnt_full, _N0P, _N1P, _K0)
    hp = _pad_rows(h[ids0] * sc0[:, None], _N1P)

    h, lg1 = _conv(hp, s1, d1, cnt1, _N1P, W1, b1, P1, no1, ni1)
    hid1 = h
    _, ids1 = lax.top_k(lg1[:_K0], _K1)
    sc1 = jax.nn.sigmoid(lg1[ids1] + pb1[0])
    map1 = _trash_map(ids1, _K1, _N1P, _N2P)
    s2, d2, cnt2, no2, ni2 = _relabel(map1, s1, d1, cnt1, _N1P, _N2P, _K1)
    hp = _pad_rows(h[ids1] * sc1[:, None], _N2P)

    h, _ = _conv(hp, s2, d2, cnt2, _N2P, W2, b2, None, no2, ni2)

    u = jnp.zeros((_N1P, _D), jnp.float32).at[ids1].set(h[:_K1]) + hid1
    u = u.at[_K0:].set(0.0)
    h, _ = _conv(u, s1, d1, cnt1, _N1P, W3, b3, None, no1, ni1)

    u = jnp.zeros((_N0P, _D), jnp.float32).at[ids0].set(h[:_K0]) + hid0
    u = u.at[_N0:].set(0.0)
    h, _ = _conv(u, s0, d0, cnt_full, _N0P, W4, b4, None, no0, ni0)
    return h[:_N0]


# speedup vs baseline: 1.0023x; 1.0023x over previous
import functools

import jax
import jax.numpy as jnp
from jax import lax
from jax.experimental import pallas as pl
from jax.experimental.pallas import tpu as pltpu
from jax.experimental.pallas import tpu_sc as plsc

_E = 320000
_E_PAD = 327680
_NW = 32
_EPT = _E_PAD // _NW
_CH = 128
_NCH = _EPT // _CH
_D = 128

_N0, _K0, _K1 = 10000, 5000, 2500
_N0P, _N1P, _N2P = 10240, 5120, 2560


@functools.cache
def _make_scatter(n_pad):
    mesh = plsc.VectorSubcoreMesh(core_axis_name="c", subcore_axis_name="s")
    rpt = n_pad // 16

    @functools.partial(
        pl.kernel,
        out_type=jax.ShapeDtypeStruct((2, n_pad, _D), jnp.float32),
        mesh=mesh,
        scratch_types=[
            pltpu.VMEM((_CH,), jnp.int32),
            pltpu.VMEM((_CH,), jnp.int32),
            pltpu.VMEM((_CH, _D), jnp.float32),
            pltpu.VMEM((_CH,), jnp.int32),
            pltpu.VMEM((_CH,), jnp.int32),
            pltpu.VMEM((_CH, _D), jnp.float32),
            pltpu.VMEM((16,), jnp.int32),
            pltpu.VMEM_SHARED((n_pad, _D), jnp.float32),
            pltpu.SemaphoreType.DMA,
            pltpu.SemaphoreType.DMA,
        ],
        compiler_params=pltpu.CompilerParams(needs_layout_passes=False),
    )
    def k(hpre_hbm, src_hbm, dst_hbm, zeros_hbm, counts_hbm, out_hbm,
          giA, siA, rowsA, giB, siB, rowsB, cbuf, acc, semA, semB):
        cid = lax.axis_index("c")
        sid = lax.axis_index("s")
        wid = sid * 2 + cid
        pltpu.sync_copy(zeros_hbm, acc.at[pl.ds(sid * rpt, rpt)])
        pltpu.sync_copy(counts_hbm.at[wid], cbuf)
        npairs = cbuf[pl.ds(0, 16)][0]
        plsc.subcore_barrier()
        base = wid * _EPT

        pltpu.sync_copy(src_hbm.at[pl.ds(base, _CH)], giA)
        pltpu.sync_copy(dst_hbm.at[pl.ds(base, _CH)], siA)
        pltpu.async_copy(hpre_hbm.at[giA], rowsA, semA)

        def body(i, carry):
            offb = base + (2 * i + 1) * _CH
            pltpu.sync_copy(src_hbm.at[pl.ds(offb, _CH)], giB)
            pltpu.sync_copy(dst_hbm.at[pl.ds(offb, _CH)], siB)
            pltpu.async_copy(hpre_hbm.at[giB], rowsB, semB)
            pltpu.make_async_copy(hpre_hbm.at[giA], rowsA, semA).wait()
            pltpu.sync_copy(rowsA, acc.at[siA], add=True)

            @pl.when(i < npairs - 1)
            def _():
                offa = base + (2 * i + 2) * _CH
                pltpu.sync_copy(src_hbm.at[pl.ds(offa, _CH)], giA)
                pltpu.sync_copy(dst_hbm.at[pl.ds(offa, _CH)], siA)
                pltpu.async_copy(hpre_hbm.at[giA], rowsA, semA)

            pltpu.make_async_copy(hpre_hbm.at[giB], rowsB, semB).wait()
            pltpu.sync_copy(rowsB, acc.at[siB], add=True)
            return carry

        lax.fori_loop(0, npairs, body, 0)
        plsc.subcore_barrier()
        pltpu.sync_copy(
            acc.at[pl.ds(sid * rpt, rpt)],
            out_hbm.at[cid].at[pl.ds(sid * rpt, rpt)],
        )

    return k


@functools.cache
def _make_relabel(n_old_pad, n_new_pad, k_new):
    mesh = plsc.VectorSubcoreMesh(core_axis_name="c", subcore_axis_name="s")
    rh = n_new_pad // 128
    R = 2 * rh

    @functools.partial(
        pl.kernel,
        out_type=[
            jax.ShapeDtypeStruct((2, _E_PAD), jnp.int32),
            jax.ShapeDtypeStruct((_NW, 16), jnp.int32),
            jax.ShapeDtypeStruct((2, R, 128), jnp.float32),
        ],
        mesh=mesh,
        scratch_types=[
            pltpu.VMEM((n_old_pad,), jnp.int32),
            pltpu.VMEM((_EPT,), jnp.int32),
            pltpu.VMEM((_EPT,), jnp.int32),
            pltpu.VMEM((_EPT + 256,), jnp.int32),
            pltpu.VMEM((_EPT + 256,), jnp.int32),
            pltpu.VMEM((R, 128), jnp.float32),
            pltpu.VMEM((min(R, 128),), jnp.int32),
            pltpu.VMEM((max(R - 128, 8),), jnp.int32),
            pltpu.VMEM((16,), jnp.int32),
            pltpu.VMEM((16,), jnp.int32),
            pltpu.VMEM_SHARED((R, 128), jnp.float32),
            pltpu.VMEM_SHARED((n_old_pad,), jnp.int32),
        ],
        compiler_params=pltpu.CompilerParams(needs_layout_passes=False),
    )
    def k(map_hbm, s_hbm, d_hbm, zeros_hbm, iota_hbm, counts_in_hbm,
          edges_hbm, counts_hbm, deg_hbm,
          mapv, sin, din, sb, db, hist, ri0, ri1, cnt16, cin16, acc, maps):
        cid = lax.axis_index("c")
        sid = lax.axis_index("s")
        wid = sid * 2 + cid
        base = wid * _EPT

        @pl.when(sid == 0)
        def _():
            pltpu.sync_copy(map_hbm, maps)

        plsc.subcore_barrier()
        pltpu.sync_copy(maps, mapv)
        pltpu.sync_copy(s_hbm.at[pl.ds(base, _EPT)], sin)
        pltpu.sync_copy(d_hbm.at[pl.ds(base, _EPT)], din)
        pltpu.sync_copy(counts_in_hbm.at[wid], cin16)
        pltpu.sync_copy(zeros_hbm, hist)

        @pl.when(sid == 0)
        def _():
            pltpu.sync_copy(zeros_hbm, acc)

        pltpu.sync_copy(iota_hbm.at[pl.ds(0, min(R, 128))], ri0)
        if R > 128:
            pltpu.sync_copy(iota_hbm.at[pl.ds(128, R - 128)], ri1)
        ones = jnp.ones((16,), jnp.float32)

        def body(i, cur):
            s16 = sin[pl.ds(i * 16, 16)]
            d16 = din[pl.ds(i * 16, 16)]
            ns = plsc.load_gather(mapv, [s16])
            nd = plsc.load_gather(mapv, [d16])
            m = (ns < k_new) & (nd < k_new)
            plsc.addupdate_scatter(
                hist, [lax.shift_right_logical(ns, 7), ns & 127], ones, mask=m)
            plsc.addupdate_scatter(
                hist, [lax.shift_right_logical(nd, 7) + rh, nd & 127], ones,
                mask=m)
            plsc.store_compressed(sb.at[pl.ds(cur, 16)], ns, mask=m)
            plsc.store_compressed(db.at[pl.ds(cur, 16)], nd, mask=m)
            return cur + plsc.all_reduce_population_count(m)[0]

        cur = lax.fori_loop(0, cin16[pl.ds(0, 16)][0] * 16, body, 0)
        lane = jnp.arange(16, dtype=jnp.int32)
        for j in range(16):
            sb[pl.ds(cur + j * 16, 16)] = jnp.zeros((16,), jnp.int32)
            db[pl.ds(cur + j * 16, 16)] = k_new + ((lane + j * 16) & 31)
        npairs = jnp.maximum(lax.shift_right_logical(cur + 255, 8), 1)
        cnt16[...] = jnp.full((16,), 0, jnp.int32) + npairs
        pltpu.sync_copy(cnt16, counts_hbm.at[wid])
        pltpu.sync_copy(sb.at[pl.ds(0, _EPT)],
                        edges_hbm.at[0].at[pl.ds(base, _EPT)])
        pltpu.sync_copy(db.at[pl.ds(0, _EPT)],
                        edges_hbm.at[1].at[pl.ds(base, _EPT)])
        plsc.subcore_barrier()
        if R <= 128:
            pltpu.sync_copy(hist, acc.at[ri0], add=True)
        else:
            pltpu.sync_copy(hist.at[pl.ds(0, 128)], acc.at[ri0], add=True)
            pltpu.sync_copy(hist.at[pl.ds(128, R - 128)], acc.at[ri1], add=True)
        plsc.subcore_barrier()

        @pl.when(sid == 0)
        def _():
            pltpu.sync_copy(acc, deg_hbm.at[cid])

    return k


@functools.cache
def _make_degrees(n_pad, n_real):
    mesh = plsc.VectorSubcoreMesh(core_axis_name="c", subcore_axis_name="s")
    rh = n_pad // 128
    R = 2 * rh

    @functools.partial(
        pl.kernel,
        out_type=jax.ShapeDtypeStruct((2, R, 128), jnp.float32),
        mesh=mesh,
        scratch_types=[
            pltpu.VMEM((_EPT,), jnp.int32),
            pltpu.VMEM((_EPT,), jnp.int32),
            pltpu.VMEM((R, 128), jnp.float32),
            pltpu.VMEM((min(R, 128),), jnp.int32),
            pltpu.VMEM((max(R - 128, 8),), jnp.int32),
            pltpu.VMEM_SHARED((R, 128), jnp.float32),
            pltpu.SemaphoreType.DMA,
        ],
        compiler_params=pltpu.CompilerParams(needs_layout_passes=False),
    )
    def k(src_hbm, dst_hbm, zeros_hbm, iota_hbm, out_hbm,
          si, di, hist, ri0, ri1, acc, sem):
        cid = lax.axis_index("c")
        sid = lax.axis_index("s")
        wid = sid * 2 + cid
        pltpu.sync_copy(zeros_hbm, hist)

        @pl.when(sid == 0)
        def _():
            pltpu.sync_copy(zeros_hbm, acc)

        base = wid * _EPT
        pltpu.sync_copy(src_hbm.at[pl.ds(base, _EPT)], si)
        pltpu.sync_copy(dst_hbm.at[pl.ds(base, _EPT)], di)
        ones = jnp.ones((16,), jnp.float32)

        def body(i, carry):
            s16 = si[pl.ds(i * 16, 16)]
            d16 = di[pl.ds(i * 16, 16)]
            m = (s16 < n_real) & (d16 < n_real)
            plsc.addupdate_scatter(
                hist, [lax.shift_right_logical(s16, 7), s16 & 127], ones, mask=m)
            plsc.addupdate_scatter(
                hist, [lax.shift_right_logical(d16, 7) + rh, d16 & 127], ones, mask=m)
            return carry

        lax.fori_loop(0, _EPT // 16, body, 0)
        pltpu.sync_copy(iota_hbm.at[pl.ds(0, min(R, 128))], ri0)
        if R > 128:
            pltpu.sync_copy(iota_hbm.at[pl.ds(128, R - 128)], ri1)
        plsc.subcore_barrier()
        if R <= 128:
            pltpu.sync_copy(hist, acc.at[ri0], add=True)
        else:
            pltpu.sync_copy(hist.at[pl.ds(0, 128)], acc.at[ri0], add=True)
            pltpu.sync_copy(hist.at[pl.ds(128, R - 128)], acc.at[ri1], add=True)
        plsc.subcore_barrier()

        @pl.when(sid == 0)
        def _():
            pltpu.sync_copy(acc, out_hbm.at[cid])

    return k


def _degrees_sc(s, d, n_pad, n_real):
    R = 2 * n_pad // 128
    zeros = jnp.zeros((R, 128), jnp.float32)
    iota = jnp.arange(R, dtype=jnp.int32)
    parts = _make_degrees(n_pad, n_real)(s, d, zeros, iota)
    deg2 = (parts[0] + parts[1]).reshape(2, n_pad)
    return deg2[0], deg2[1]


@functools.cache
def _make_dense(n_pad, with_scores):
    br = 256
    grid = (n_pad // br,)

    def body(agg_ref, ni_ref, w_ref, b_ref, p_ref, h_ref, s_ref):
        a = (agg_ref[0] + agg_ref[1]) * ni_ref[...]
        h = jnp.dot(a, w_ref[...], preferred_element_type=jnp.float32)
        h = jnp.maximum(h + b_ref[...], 0.0)
        h_ref[...] = h
        if with_scores:
            s_ref[...] = jnp.dot(h, p_ref[...], preferred_element_type=jnp.float32)

    def body_ns(agg_ref, ni_ref, w_ref, b_ref, p_ref, h_ref):
        body(agg_ref, ni_ref, w_ref, b_ref, p_ref, h_ref, None)

    out_shape = [jax.ShapeDtypeStruct((n_pad, _D), jnp.float32)]
    out_specs = [pl.BlockSpec((br, _D), lambda i: (i, 0))]
    if with_scores:
        out_shape.append(jax.ShapeDtypeStruct((n_pad, 1), jnp.float32))
        out_specs.append(pl.BlockSpec((br, 1), lambda i: (i, 0)))

    return pl.pallas_call(
        body if with_scores else body_ns,
        grid=grid,
        in_specs=[
            pl.BlockSpec((2, br, _D), lambda i: (0, i, 0)),
            pl.BlockSpec((br, 1), lambda i: (i, 0)),
            pl.BlockSpec((_D, _D), lambda i: (0, 0)),
            pl.BlockSpec((1, _D), lambda i: (0, 0)),
            pl.BlockSpec((_D, 1), lambda i: (0, 0)),
        ],
        out_specs=out_specs,
        out_shape=out_shape,
    )


def _deg_to_norms(parts, n_pad):
    deg2 = (parts[0] + parts[1]).reshape(2, n_pad)
    dego, degi = deg2[0], deg2[1]
    norm_out = jnp.where(dego > 0, lax.rsqrt(jnp.maximum(dego, 1.0)), 0.0)
    norm_in = jnp.where(degi > 0, lax.rsqrt(jnp.maximum(degi, 1.0)), 0.0)
    return norm_out, norm_in


def _norms(s, d, n_pad, n_real):
    R = 2 * n_pad // 128
    zeros = jnp.zeros((R, 128), jnp.float32)
    iota = jnp.arange(R, dtype=jnp.int32)
    parts = _make_degrees(n_pad, n_real)(s, d, zeros, iota)
    return _deg_to_norms(parts, n_pad)


def _relabel(map_ext, s, d, counts_in, n_old_pad, n_new_pad, k_new):
    R = 2 * n_new_pad // 128
    zeros = jnp.zeros((R, 128), jnp.float32)
    iota = jnp.arange(R, dtype=jnp.int32)
    edges, counts, degp = _make_relabel(n_old_pad, n_new_pad, k_new)(
        map_ext, s, d, zeros, iota, counts_in)
    no, ni = _deg_to_norms(degp, n_new_pad)
    return edges[0], edges[1], counts, no, ni


def _conv(h, s, d, counts, n_pad, W, b, P, norm_out, norm_in):
    h_pre = h * norm_out[:, None]
    zeros = jnp.zeros((n_pad // 16, _D), jnp.float32)
    agg2 = _make_scatter(n_pad)(h_pre, s, d, zeros, counts)
    dense = _make_dense(n_pad, P is not None)
    pdummy = jnp.zeros((_D, 1), jnp.float32) if P is None else P
    outs = dense(agg2, norm_in[:, None], W, b.reshape(1, _D), pdummy)
    if P is None:
        return outs[0], None
    return outs[0], outs[1][:, 0]


def _pad_rows(h, n_pad):
    return jnp.pad(h, ((0, n_pad - h.shape[0]), (0, 0)))


def _trash_map(ids, k_new, n_old_pad, n_new_pad):
    base = k_new + (jnp.arange(n_old_pad, dtype=jnp.int32) % (n_new_pad - k_new))
    return base.at[ids].set(jnp.arange(k_new, dtype=jnp.int32))


def kernel(x, edge_index, W0, b0, W1, b1, W2, b2, W3, b3, W4, b4, P0, pb0, P1, pb1):
    src = edge_index[0]
    dst = edge_index[1]
    pad = _N0 + (jnp.arange(_E_PAD - _E, dtype=jnp.int32) % (_N0P - _N0))
    s0 = jnp.concatenate([src, pad])
    d0 = jnp.concatenate([dst, pad])

    cnt_full = jnp.full((_NW, 16), _NCH // 2, jnp.int32)

    no0, ni0 = _norms(s0, d0, _N0P, _N0)
    h, lg0 = _conv(_pad_rows(x, _N0P), s0, d0, cnt_full, _N0P, W0, b0, P0,
                   no0, ni0)
    hid0 = h
    _, ids0 = lax.top_k(lg0[:_N0], _K0)
    sc0 = jax.nn.sigmoid(lg0[ids0] + pb0[0])
    map0 = _trash_map(ids0, _K0, _N0P, _N1P)
    s1, d1, cnt1, no1, ni1 = _relabel(map0, s0, d0, cnt_full, _N0P, _N1P, _K0)
    hp = _pad_rows(h[ids0] * sc0[:, None], _N1P)

    h, lg1 = _conv(hp, s1, d1, cnt1, _N1P, W1, b1, P1, no1, ni1)
    hid1 = h
    _, ids1 = lax.top_k(lg1[:_K0], _K1)
    sc1 = jax.nn.sigmoid(lg1[ids1] + pb1[0])
    map1 = _trash_map(ids1, _K1, _N1P, _N2P)
    s2, d2, cnt2, no2, ni2 = _relabel(map1, s1, d1, cnt1, _N1P, _N2P, _K1)
    hp = _pad_rows(h[ids1] * sc1[:, None], _N2P)

    h, _ = _conv(hp, s2, d2, cnt2, _N2P, W2, b2, None, no2, ni2)

    u = jnp.zeros((_N1P, _D), jnp.float32).at[ids1].set(h[:_K1]) + hid1
    u = u.at[_K0:].set(0.0)
    h, _ = _conv(u, s1, d1, cnt1, _N1P, W3, b3, None, no1, ni1)

    u = jnp.zeros((_N0P, _D), jnp.float32).at[ids0].set(h[:_K0]) + hid0
    u = u.at[_N0:].set(0.0)
    h, _ = _conv(u, s0, d0, cnt_full, _N0P, W4, b4, None, no0, ni0)
    return h[:_N0]
